# baseline (speedup 1.0000x reference)
import jax
import jax.numpy as jnp
from jax.experimental import pallas as pl
from jax.experimental.pallas import tpu as pltpu
from jax.experimental.pallas import tpu_sc as plsc

NUM_ROWS = 1000000
BATCH = 16384
DIM = 64
WINDOW = 128
LANES = 16


def kernel(user, item, user_table, item_table):
    utT = user_table.T
    itT = item_table.T
    u2 = user.reshape(1, BATCH)
    i2 = item.reshape(1, BATCH)

    mesh = plsc.VectorSubcoreMesh(core_axis_name="core",
                                  subcore_axis_name="subcore")

    @pl.kernel(
        out_type=jax.ShapeDtypeStruct((DIM, BATCH), jnp.float32),
        mesh=mesh,
        compiler_params=pltpu.CompilerParams(use_tc_tiling_on_sc=False),
        scratch_types=[
            pltpu.VMEM((DIM, WINDOW), jnp.float32),
            pltpu.VMEM((DIM, WINDOW), jnp.float32),
            pltpu.SemaphoreType.DMA,
            pltpu.SemaphoreType.DMA,
        ],
    )
    def sc_kernel(u_hbm, i_hbm, ut_hbm, it_hbm, o_hbm, ubuf, ibuf, sem_u, sem_i):
        def body(u_idx, i_idx, o_vmem):
            @pl.loop(0, DIM)
            def _(d):
                pltpu.make_async_copy(
                    ut_hbm.at[d].at[u_idx.at[0]], ubuf.at[d], sem_u).start()
                pltpu.make_async_copy(
                    it_hbm.at[d].at[i_idx.at[0]], ibuf.at[d], sem_i).start()

            @pl.loop(0, DIM)
            def _(d):
                pltpu.make_async_copy(
                    ut_hbm.at[d].at[u_idx.at[0]], ubuf.at[d], sem_u).wait()
                pltpu.make_async_copy(
                    it_hbm.at[d].at[i_idx.at[0]], ibuf.at[d], sem_i).wait()

            @pl.loop(0, DIM)
            def _(d):
                @pl.loop(0, WINDOW, step=LANES)
                def _(c):
                    slc = (pl.ds(d, 1), pl.ds(c, LANES))
                    o_vmem.at[*slc][...] = ubuf.at[*slc][...] * ibuf.at[*slc][...]

        pltpu.emit_pipeline(
            body,
            grid=(BATCH // WINDOW,),
            in_specs=[
                pl.BlockSpec((1, WINDOW), lambda i: (0, i)),
                pl.BlockSpec((1, WINDOW), lambda i: (0, i)),
            ],
            out_specs=[pl.BlockSpec((DIM, WINDOW), lambda i: (0, i))],
            core_axis_name=("core", "subcore"),
            dimension_semantics=(pltpu.PARALLEL,),
        )(u_hbm, i_hbm, o_hbm)

    out = sc_kernel(u2, i2, utT, itT)
    return out.T
